# baseline (speedup 1.0000x reference)
import functools

import jax
import jax.numpy as jnp
from jax import lax
from jax.experimental import pallas as pl
from jax.experimental.pallas import tpu as pltpu
from jax.experimental.pallas import tpu_sc as plsc

N = 512
NCELLS = N * N
NUM_STEPS = 10
NW = 32
CPW = NCELLS // NW
LANES = 16
VPW = CPW // LANES
UNROLL = 8


def _make_advect(nf, coords_in_fields):
  mesh = plsc.VectorSubcoreMesh(core_axis_name="c", subcore_axis_name="s")
  out_type = [jax.ShapeDtypeStruct((NCELLS,), jnp.float32) for _ in range(nf)]
  scratch_types = (
      [pltpu.VMEM((CPW,), jnp.float32) for _ in range(2)]
      + [pltpu.VMEM((CPW,), jnp.int32) for _ in range(4)]
      + [pltpu.VMEM((CPW,), jnp.float32) for _ in range(2)]
      + [pltpu.VMEM((CPW,), jnp.float32) for _ in range(4)]
      + [pltpu.VMEM((CPW,), jnp.float32) for _ in range(2)]
      + [pltpu.VMEM_SHARED((NCELLS,), jnp.float32)]
      + [pltpu.SemaphoreType.DMA, pltpu.SemaphoreType.DMA,
         pltpu.SemaphoreType.DMA]
  )
  ncoord = 0 if coords_in_fields else 2

  @functools.partial(
      pl.kernel, mesh=mesh, out_type=out_type, scratch_types=scratch_types,
      compiler_params=pltpu.CompilerParams(needs_layout_passes=False))
  def advect_kernel(*refs):
    f_hbm = refs[ncoord:ncoord + nf]
    o_hbm = refs[ncoord + nf:ncoord + 2 * nf]
    (vx_v, vy_v, i00, i01, i10, i11, rw_v, bw_v,
     g00, g01, g10, g11, bb0, bb1, f_sh, sem, sem2,
     sem3) = refs[ncoord + 2 * nf:]
    if coords_in_fields:
      cvx_hbm, cvy_hbm = f_hbm[-2], f_hbm[-1]
    else:
      cvx_hbm, cvy_hbm = refs[0], refs[1]
    sid = lax.axis_index("s")
    wid = sid * 2 + lax.axis_index("c")
    base = wid * CPW
    pltpu.sync_copy(cvx_hbm.at[pl.ds(base, CPW)], vx_v)
    pltpu.sync_copy(cvy_hbm.at[pl.ds(base, CPW)], vy_v)
    spc2 = NCELLS // 16

    def fire_h1(fi):
      return [pltpu.async_copy(
          f_hbm[fi].at[pl.ds(sid * spc2 + j * CPW, CPW)],
          (bb0, bb1)[j], sem2) for j in range(2)]

    def drain_h2(cps):
      for cp in cps:
        cp.wait()
      h2 = [pltpu.async_copy(
          (bb0, bb1)[j], f_sh.at[pl.ds(sid * spc2 + j * CPW, CPW)],
          sem2) for j in range(2)]
      for cp in h2:
        cp.wait()

    h1 = fire_h1(0)

    lanes = lax.iota(jnp.int32, LANES)
    lanes_f = lanes.astype(jnp.float32)
    row0 = wid * 16
    spc = NCELLS // 16

    @plsc.parallel_loop(0, VPW, unroll=UNROLL)
    def index_body(k):
      off = k * LANES
      s = pl.ds(off, LANES)
      i_f = (row0 + lax.shift_right_logical(k, 5)).astype(jnp.float32)
      jb_f = (lax.bitwise_and(k, 31) * 16).astype(jnp.float32)
      cx = i_f - vx_v[s]
      cy = (jb_f + lanes_f) - vy_v[s]
      ti = cx.astype(jnp.int32)
      r0 = ti - (ti.astype(jnp.float32) > cx).astype(jnp.int32)
      rw_v[s] = cx - r0.astype(jnp.float32)
      tj = cy.astype(jnp.int32)
      c0 = tj - (tj.astype(jnp.float32) > cy).astype(jnp.int32)
      bw_v[s] = cy - c0.astype(jnp.float32)
      r0m = lax.bitwise_and(r0, 511)
      r1m = lax.bitwise_and(r0m + 1, 511)
      c0m = lax.bitwise_and(c0, 511)
      c1m = lax.bitwise_and(c0m + 1, 511)
      r0s = lax.shift_left(r0m, 9)
      r1s = lax.shift_left(r1m, 9)
      i00[s] = lax.bitwise_or(r0s, c0m)
      i01[s] = lax.bitwise_or(r0s, c1m)
      i10[s] = lax.bitwise_or(r1s, c0m)
      i11[s] = lax.bitwise_or(r1s, c1m)

    drain_h2(h1)
    plsc.subcore_barrier()

    half = CPW // 2

    def fire_half(h, s_):
      sl = pl.ds(h * half, half)
      return [pltpu.async_copy(f_sh.at[i00.at[sl]], g00.at[sl], s_),
              pltpu.async_copy(f_sh.at[i01.at[sl]], g01.at[sl], s_),
              pltpu.async_copy(f_sh.at[i10.at[sl]], g10.at[sl], s_),
              pltpu.async_copy(f_sh.at[i11.at[sl]], g11.at[sl], s_)]

    def combine(lo, hi):
      @plsc.parallel_loop(lo, hi, unroll=UNROLL)
      def combine_body(k):
        s = pl.ds(k * LANES, LANES)
        rw = rw_v[s]
        bw = bw_v[s]
        top = (1.0 - bw) * g00[s] + bw * g01[s]
        bot = (1.0 - bw) * g10[s] + bw * g11[s]
        vx_v[s] = (1.0 - rw) * top + rw * bot

    for fi in range(nf):
      c0 = fire_half(0, sem)
      c1 = fire_half(1, sem3)
      for cp in c0:
        cp.wait()
      if fi + 1 < nf:
        h1 = fire_h1(fi + 1)
      combine(0, VPW // 2)
      for cp in c1:
        cp.wait()
      combine(VPW // 2, VPW)
      pltpu.sync_copy(vx_v, o_hbm[fi].at[pl.ds(base, CPW)])
      if fi + 1 < nf:
        plsc.subcore_barrier()
        drain_h2(h1)
        plsc.subcore_barrier()

  return advect_kernel


_advect1 = _make_advect(1, coords_in_fields=False)
_advect2 = _make_advect(2, coords_in_fields=True)
_advect3 = _make_advect(3, coords_in_fields=True)


def _roll(x, shift, axis):
  if axis == 0:
    if shift == 1:
      return jnp.concatenate([x[-1:, :], x[:-1, :]], axis=0)
    return jnp.concatenate([x[1:, :], x[:1, :]], axis=0)
  if shift == 1:
    return jnp.concatenate([x[:, -1:], x[:, :-1]], axis=1)
  return jnp.concatenate([x[:, 1:], x[:, :1]], axis=1)


def _project_body(vx_ref, vy_ref, vxo_ref, vyo_ref):
  vx = vx_ref[...]
  vy = vy_ref[...]
  h = 1.0 / N
  div = -0.5 * h * (_roll(vx, -1, 0) - _roll(vx, 1, 0)
                    + _roll(vy, -1, 1) - _roll(vy, 1, 1))
  p = jnp.zeros_like(div)
  for _ in range(10):
    p = (div + _roll(p, 1, 0) + _roll(p, -1, 0)
         + _roll(p, 1, 1) + _roll(p, -1, 1)) / 4.0
  vxo_ref[...] = vx - 0.5 * (_roll(p, -1, 0) - _roll(p, 1, 0)) / h
  vyo_ref[...] = vy - 0.5 * (_roll(p, -1, 1) - _roll(p, 1, 1)) / h


_project = pl.pallas_call(
    _project_body,
    out_shape=[jax.ShapeDtypeStruct((N, N), jnp.float32) for _ in range(2)],
)


def kernel(smoke, init_vx, init_vy):
  sf = smoke.reshape(-1)

  ax, ay = _advect2(init_vx.reshape(-1), init_vy.reshape(-1))
  vx, vy = _project(ax.reshape(N, N), ay.reshape(N, N))

  for _ in range(NUM_STEPS - 1):
    sf, ax, ay = _advect3(sf, vx.reshape(-1), vy.reshape(-1))
    vx, vy = _project(ax.reshape(N, N), ay.reshape(N, N))

  (sf,) = _advect1(vx.reshape(-1), vy.reshape(-1), sf)
  return sf.reshape(N, N)


# speedup vs baseline: 1.0098x; 1.0098x over previous
import functools

import jax
import jax.numpy as jnp
from jax import lax
from jax.experimental import pallas as pl
from jax.experimental.pallas import tpu as pltpu
from jax.experimental.pallas import tpu_sc as plsc

N = 512
NCELLS = N * N
NUM_STEPS = 10
NW = 32
CPW = NCELLS // NW
LANES = 16
VPW = CPW // LANES
UNROLL = 8


def _make_advect(nf, coords_in_fields):
  mesh = plsc.VectorSubcoreMesh(core_axis_name="c", subcore_axis_name="s")
  out_type = [jax.ShapeDtypeStruct((NCELLS,), jnp.float32) for _ in range(nf)]
  scratch_types = (
      [pltpu.VMEM((CPW,), jnp.float32) for _ in range(2)]
      + [pltpu.VMEM((CPW,), jnp.int32) for _ in range(4)]
      + [pltpu.VMEM((CPW,), jnp.float32) for _ in range(2)]
      + [pltpu.VMEM((CPW,), jnp.float32) for _ in range(4)]
      + [pltpu.VMEM((CPW,), jnp.float32) for _ in range(2)]
      + [pltpu.VMEM_SHARED((NCELLS,), jnp.float32)]
      + [pltpu.SemaphoreType.DMA, pltpu.SemaphoreType.DMA]
  )
  ncoord = 0 if coords_in_fields else 2

  @functools.partial(
      pl.kernel, mesh=mesh, out_type=out_type, scratch_types=scratch_types,
      compiler_params=pltpu.CompilerParams(needs_layout_passes=False))
  def advect_kernel(*refs):
    f_hbm = refs[ncoord:ncoord + nf]
    o_hbm = refs[ncoord + nf:ncoord + 2 * nf]
    (vx_v, vy_v, i00, i01, i10, i11, rw_v, bw_v,
     g00, g01, g10, g11, bb0, bb1, f_sh, sem, sem2) = refs[ncoord + 2 * nf:]
    if coords_in_fields:
      cvx_hbm, cvy_hbm = f_hbm[-2], f_hbm[-1]
    else:
      cvx_hbm, cvy_hbm = refs[0], refs[1]
    sid = lax.axis_index("s")
    wid = sid * 2 + lax.axis_index("c")
    base = wid * CPW
    pltpu.sync_copy(cvx_hbm.at[pl.ds(base, CPW)], vx_v)
    pltpu.sync_copy(cvy_hbm.at[pl.ds(base, CPW)], vy_v)
    spc2 = NCELLS // 16

    def fire_h1(fi):
      return [pltpu.async_copy(
          f_hbm[fi].at[pl.ds(sid * spc2 + j * CPW, CPW)],
          (bb0, bb1)[j], sem2) for j in range(2)]

    def drain_h2(cps):
      for cp in cps:
        cp.wait()
      h2 = [pltpu.async_copy(
          (bb0, bb1)[j], f_sh.at[pl.ds(sid * spc2 + j * CPW, CPW)],
          sem2) for j in range(2)]
      for cp in h2:
        cp.wait()

    h1 = fire_h1(0)

    lanes = lax.iota(jnp.int32, LANES)
    lanes_f = lanes.astype(jnp.float32)
    row0 = wid * 16
    spc = NCELLS // 16

    @plsc.parallel_loop(0, VPW, unroll=UNROLL)
    def index_body(k):
      off = k * LANES
      s = pl.ds(off, LANES)
      i_f = (row0 + lax.shift_right_logical(k, 5)).astype(jnp.float32)
      jb_f = (lax.bitwise_and(k, 31) * 16).astype(jnp.float32)
      cx = i_f - vx_v[s]
      cy = (jb_f + lanes_f) - vy_v[s]
      ti = cx.astype(jnp.int32)
      r0 = ti - (ti.astype(jnp.float32) > cx).astype(jnp.int32)
      rw_v[s] = cx - r0.astype(jnp.float32)
      tj = cy.astype(jnp.int32)
      c0 = tj - (tj.astype(jnp.float32) > cy).astype(jnp.int32)
      bw_v[s] = cy - c0.astype(jnp.float32)
      r0m = lax.bitwise_and(r0, 511)
      r1m = lax.bitwise_and(r0m + 1, 511)
      c0m = lax.bitwise_and(c0, 511)
      c1m = lax.bitwise_and(c0m + 1, 511)
      r0s = lax.shift_left(r0m, 9)
      r1s = lax.shift_left(r1m, 9)
      i00[s] = lax.bitwise_or(r0s, c0m)
      i01[s] = lax.bitwise_or(r0s, c1m)
      i10[s] = lax.bitwise_or(r1s, c0m)
      i11[s] = lax.bitwise_or(r1s, c1m)

    drain_h2(h1)
    plsc.subcore_barrier()

    for fi in range(nf):
      cps = [pltpu.async_copy(f_sh.at[i00], g00, sem),
             pltpu.async_copy(f_sh.at[i01], g01, sem),
             pltpu.async_copy(f_sh.at[i10], g10, sem),
             pltpu.async_copy(f_sh.at[i11], g11, sem)]
      for cp in cps:
        cp.wait()
      if fi + 1 < nf:
        h1 = fire_h1(fi + 1)

      @plsc.parallel_loop(0, VPW, unroll=UNROLL)
      def combine_body(k):
        s = pl.ds(k * LANES, LANES)
        rw = rw_v[s]
        bw = bw_v[s]
        top = (1.0 - bw) * g00[s] + bw * g01[s]
        bot = (1.0 - bw) * g10[s] + bw * g11[s]
        vx_v[s] = (1.0 - rw) * top + rw * bot

      pltpu.sync_copy(vx_v, o_hbm[fi].at[pl.ds(base, CPW)])
      if fi + 1 < nf:
        plsc.subcore_barrier()
        drain_h2(h1)
        plsc.subcore_barrier()

  return advect_kernel


_advect1 = _make_advect(1, coords_in_fields=False)
_advect2 = _make_advect(2, coords_in_fields=True)
_advect3 = _make_advect(3, coords_in_fields=True)


def _roll(x, shift, axis):
  if axis == 0:
    if shift == 1:
      return jnp.concatenate([x[-1:, :], x[:-1, :]], axis=0)
    return jnp.concatenate([x[1:, :], x[:1, :]], axis=0)
  if shift == 1:
    return jnp.concatenate([x[:, -1:], x[:, :-1]], axis=1)
  return jnp.concatenate([x[:, 1:], x[:, :1]], axis=1)


def _project_body(vx_ref, vy_ref, vxo_ref, vyo_ref):
  vx = vx_ref[...]
  vy = vy_ref[...]
  h = 1.0 / N
  div = -0.5 * h * (_roll(vx, -1, 0) - _roll(vx, 1, 0)
                    + _roll(vy, -1, 1) - _roll(vy, 1, 1))
  p = jnp.zeros_like(div)
  for _ in range(10):
    p = (div + _roll(p, 1, 0) + _roll(p, -1, 0)
         + _roll(p, 1, 1) + _roll(p, -1, 1)) / 4.0
  vxo_ref[...] = vx - 0.5 * (_roll(p, -1, 0) - _roll(p, 1, 0)) / h
  vyo_ref[...] = vy - 0.5 * (_roll(p, -1, 1) - _roll(p, 1, 1)) / h


_project = pl.pallas_call(
    _project_body,
    out_shape=[jax.ShapeDtypeStruct((N, N), jnp.float32) for _ in range(2)],
)


def kernel(smoke, init_vx, init_vy):
  sf = smoke.reshape(-1)

  ax, ay = _advect2(init_vx.reshape(-1), init_vy.reshape(-1))
  vx, vy = _project(ax.reshape(N, N), ay.reshape(N, N))

  for _ in range(NUM_STEPS - 1):
    sf, ax, ay = _advect3(sf, vx.reshape(-1), vy.reshape(-1))
    vx, vy = _project(ax.reshape(N, N), ay.reshape(N, N))

  (sf,) = _advect1(vx.reshape(-1), vy.reshape(-1), sf)
  return sf.reshape(N, N)
